# Initial kernel scaffold; baseline (speedup 1.0000x reference)
import jax
import jax.numpy as jnp
from jax.experimental import pallas as pl


def kernel(x, edge_index, edge_weights, W_self, W_neigh, b, W_self_rev, W_neigh_rev, b_rev):
    raise NotImplementedError("write your pallas kernel here")


# speedup vs baseline: 2.2423x; 2.2423x over previous
import functools

import jax
import jax.numpy as jnp
from jax import lax
from jax.experimental import pallas as pl
from jax.experimental.pallas import tpu as pltpu
from jax.experimental.pallas import tpu_sc as plsc

N = 10000
E = 160000
D = 256
H = 128
NCORE = 2
NTILE = 16
EPT = E // NTILE
CH = 80
NCHUNK = EPT // CH
NPAD = 10240
RPT = NPAD // NTILE


def _sc_body(xcat, esrc, edst, ew, zmain, sums, degs,
             gidx_v, sidx_v, ew_v, rows_v, scaled_v, acc, sem):
    c = lax.axis_index("c")
    s = lax.axis_index("s")
    coff_v = jnp.full((16,), c * N, dtype=jnp.int32)
    rs = s * RPT
    ebase0 = s * EPT

    def do_pass(d, g_ref, s_ref):
        pltpu.sync_copy(zmain.at[pl.ds(rs, RPT)], acc.at[pl.ds(rs, RPT)])
        plsc.subcore_barrier()

        def chunk(ci, carry):
            eb = ebase0 + ci * CH
            pltpu.sync_copy(g_ref.at[pl.ds(eb, CH)], gidx_v)
            pltpu.sync_copy(s_ref.at[pl.ds(eb, CH)], sidx_v)
            pltpu.sync_copy(ew.at[pl.ds(eb, CH)], ew_v)
            for g in range(CH // 16):
                gidx_v[pl.ds(g * 16, 16)] = gidx_v[pl.ds(g * 16, 16)] + coff_v
            pltpu.async_copy(xcat.at[gidx_v], rows_v, sem).wait()

            def scale_group(g, carry2):
                ew16 = ew_v[pl.ds(g * 16, 16)]
                for e in range(16):
                    r = g * 16 + e
                    w = ew16[e]
                    for j in range(H // 16):
                        scaled_v[r, pl.ds(j * 16, 16)] = (
                            rows_v[r, pl.ds(j * 16, 16)] * w)
                return carry2

            lax.fori_loop(0, CH // 16, scale_group, 0)
            pltpu.sync_copy(scaled_v, acc.at[sidx_v], add=True)
            return carry

        lax.fori_loop(0, NCHUNK, chunk, 0)
        plsc.subcore_barrier()
        pltpu.sync_copy(acc.at[pl.ds(rs, RPT)], sums.at[d, c, pl.ds(rs, RPT)])

    do_pass(0, esrc, edst)
    do_pass(1, edst, esrc)

    one16 = jnp.ones((16,), jnp.float32)
    zero16 = jnp.zeros((16,), jnp.float32)
    for r in range(CH):
        for j in range(H // 16):
            rows_v[r, pl.ds(j * 16, 16)] = one16 if j < 4 else zero16
            scaled_v[r, pl.ds(j * 16, 16)] = zero16 if j < 4 else one16
    pltpu.sync_copy(zmain.at[pl.ds(rs, RPT)], acc.at[pl.ds(rs, RPT)])
    plsc.subcore_barrier()

    def chunk3(ci, carry):
        eb = ebase0 + ci * CH
        pltpu.sync_copy(edst.at[pl.ds(eb, CH)], sidx_v)
        pltpu.sync_copy(esrc.at[pl.ds(eb, CH)], gidx_v)
        pltpu.sync_copy(rows_v, acc.at[sidx_v], add=True)
        pltpu.sync_copy(scaled_v, acc.at[gidx_v], add=True)
        return carry

    lax.fori_loop(0, NCHUNK, chunk3, 0)
    plsc.subcore_barrier()
    pltpu.sync_copy(acc.at[pl.ds(rs, RPT)], degs.at[c, pl.ds(rs, RPT)])


def _sc_aggregate(xcat, esrc, edst, ew):
    zmain = jnp.zeros((NPAD, H), jnp.float32)
    mesh = plsc.VectorSubcoreMesh(
        core_axis_name="c", subcore_axis_name="s",
        num_cores=NCORE, num_subcores=NTILE)
    call = pl.kernel(
        _sc_body,
        out_type=[
            jax.ShapeDtypeStruct((2, NCORE, NPAD, H), jnp.float32),
            jax.ShapeDtypeStruct((NCORE, NPAD, H), jnp.float32),
        ],
        mesh=mesh,
        scratch_types=[
            pltpu.VMEM((CH,), jnp.int32),
            pltpu.VMEM((CH,), jnp.int32),
            pltpu.VMEM((CH,), jnp.float32),
            pltpu.VMEM((CH, H), jnp.float32),
            pltpu.VMEM((CH, H), jnp.float32),
            pltpu.VMEM_SHARED((NPAD, H), jnp.float32),
            pltpu.SemaphoreType.DMA,
        ],
    )
    return call(xcat, esrc, edst, ew, zmain)


R = 1024


def _tc_body(x_r, f0_r, f1_r, r0_r, r1_r, dg_r,
             ws_r, wn_r, wsr_r, wnr_r, b_r, br_r, o_r):
    dn = (((1,), (1,)), ((), ()))
    mm = functools.partial(lax.dot_general, dimension_numbers=dn,
                           preferred_element_type=jnp.float32)
    x = x_r[...]
    wn = wn_r[...]
    wnr = wnr_r[...]
    dg = dg_r[0]
    invf = 1.0 / jnp.maximum(dg[:, 0:1], 1.0)
    invr = 1.0 / jnp.maximum(dg[:, 64:65], 1.0)
    t1 = (mm(x, ws_r[...])
          + mm(f0_r[0, 0] * invf, wn[:, :H])
          + mm(f1_r[0, 0] * invf, wn[:, H:])
          + b_r[...])
    t2 = (mm(x, wsr_r[...])
          + mm(r0_r[0, 0] * invr, wnr[:, :H])
          + mm(r1_r[0, 0] * invr, wnr[:, H:])
          + br_r[...])
    o_r[...] = x + jnp.maximum(t1, 0.0) + jnp.maximum(t2, 0.0)


def _tc_combine(xp, sums, degs, W_self, W_neigh, b, W_self_rev, W_neigh_rev, b_rev):
    grid = (NPAD // R,)
    sum_spec = lambda d, c: pl.BlockSpec((1, 1, R, H), lambda i, d=d, c=c: (d, c, i, 0))
    deg_spec = pl.BlockSpec((1, R, H), lambda i: (0, i, 0))
    w_spec = pl.BlockSpec((D, D), lambda i: (0, 0))
    b_spec = pl.BlockSpec((1, D), lambda i: (0, 0))
    return pl.pallas_call(
        _tc_body,
        grid=grid,
        in_specs=[
            pl.BlockSpec((R, D), lambda i: (i, 0)),
            sum_spec(0, 0), sum_spec(0, 1), sum_spec(1, 0), sum_spec(1, 1),
            deg_spec,
            w_spec, w_spec, w_spec, w_spec, b_spec, b_spec,
        ],
        out_specs=pl.BlockSpec((R, D), lambda i: (i, 0)),
        out_shape=jax.ShapeDtypeStruct((NPAD, D), jnp.float32),
        compiler_params=pltpu.CompilerParams(
            dimension_semantics=("arbitrary",)),
    )(xp, sums, sums, sums, sums, degs,
      W_self, W_neigh, W_self_rev, W_neigh_rev,
      b.reshape(1, D), b_rev.reshape(1, D))


def kernel(x, edge_index, edge_weights, W_self, W_neigh, b,
           W_self_rev, W_neigh_rev, b_rev):
    x = x.astype(jnp.float32)
    eidx = edge_index.astype(jnp.int32)
    ew = edge_weights.astype(jnp.float32)
    xcat = jnp.concatenate([x[:, :H], x[:, H:]], axis=0)
    sums, degs = _sc_aggregate(xcat, eidx[0], eidx[1], ew)
    xp = jnp.pad(x, ((0, NPAD - N), (0, 0)))
    out = _tc_combine(xp, sums, degs, W_self, W_neigh, b,
                      W_self_rev, W_neigh_rev, b_rev)
    return out[:N]
